# Initial kernel scaffold; baseline (speedup 1.0000x reference)
import jax
import jax.numpy as jnp
from jax.experimental import pallas as pl


def kernel(x, neighbors, weights):
    raise NotImplementedError("write your pallas kernel here")


# speedup vs baseline: 3.8403x; 3.8403x over previous
import dataclasses
import functools

import jax
import jax.numpy as jnp
from jax import lax
from jax.experimental import pallas as pl
from jax.experimental.pallas import tpu as pltpu
from jax.experimental.pallas import tpu_sc as plsc

N = 10000
K = 32
D = 128
LANES = 16
NUM_WORKERS = 32
C = 8
CHUNKS = N // C
IDX_ROWS = (C * K) // 128
ITERS = (CHUNKS + NUM_WORKERS - 1) // NUM_WORKERS
G = D // LANES


def kernel(x, neighbors, weights):
    nb = neighbors.astype(jnp.int32).reshape(N * K // 128, 128)
    w = weights.reshape(N * K // 128, 128)

    mesh = plsc.VectorSubcoreMesh(core_axis_name="core", subcore_axis_name="sub")
    cp = pltpu.CompilerParams()
    if "needs_layout_passes" in pltpu.CompilerParams.__dataclass_fields__:
        cp = dataclasses.replace(cp, needs_layout_passes=False)

    @functools.partial(
        pl.kernel,
        out_type=jax.ShapeDtypeStruct((N, D), jnp.float32),
        mesh=mesh,
        compiler_params=cp,
        scratch_types=[
            pltpu.VMEM((IDX_ROWS, 128), jnp.int32),
            pltpu.VMEM((IDX_ROWS, 128), jnp.float32),
            pltpu.VMEM((IDX_ROWS, 128, D), jnp.float32),
            pltpu.VMEM((C, D), jnp.float32),
            pltpu.SemaphoreType.DMA,
        ],
    )
    def sc_kernel(x_hbm, nb_hbm, w_hbm, out_hbm, idx_v, w_v, rows_v, out_v, sem):
        wid = lax.axis_index("core") * 16 + lax.axis_index("sub")

        @pl.loop(0, ITERS)
        def _chunk(j):
            c = wid + j * NUM_WORKERS

            @pl.when(c < CHUNKS)
            def _():
                pltpu.sync_copy(nb_hbm.at[pl.ds(c * IDX_ROWS, IDX_ROWS)], idx_v)
                pltpu.sync_copy(w_hbm.at[pl.ds(c * IDX_ROWS, IDX_ROWS)], w_v)
                cp0 = pltpu.async_copy(x_hbm.at[idx_v.at[0]], rows_v.at[0], sem)
                cp1 = pltpu.async_copy(x_hbm.at[idx_v.at[1]], rows_v.at[1], sem)
                cp0.wait()
                cp1.wait()

                @pl.loop(0, C)
                def _node(i):
                    half = i // (C // IDX_ROWS)
                    r0 = (i % (C // IDX_ROWS)) * K
                    wv0 = w_v[half, pl.ds(r0, LANES)]
                    wv1 = w_v[half, pl.ds(r0 + LANES, LANES)]
                    total = jnp.sum(wv0 + wv1)
                    tot_v = jnp.broadcast_to(total, (LANES,))
                    zero_v = tot_v == 0.0
                    inv_v = jnp.where(zero_v, jnp.float32(1.0 / K), 1.0 / tot_v)
                    w0n = jnp.where(zero_v, inv_v, wv0 * inv_v)
                    w1n = jnp.where(zero_v, inv_v, wv1 * inv_v)
                    acc = [jnp.zeros((LANES,), jnp.float32) for _ in range(G)]
                    for k in range(K):
                        wk = w0n[k] if k < LANES else w1n[k - LANES]
                        for g in range(G):
                            acc[g] = acc[g] + wk * rows_v[half, r0 + k, pl.ds(g * LANES, LANES)]
                    for g in range(G):
                        out_v[i, pl.ds(g * LANES, LANES)] = acc[g]

                pltpu.sync_copy(out_v, out_hbm.at[pl.ds(c * C, C)])

    return sc_kernel(x, nb, w)


# speedup vs baseline: 4.6583x; 1.2130x over previous
import dataclasses
import functools

import jax
import jax.numpy as jnp
from jax import lax
from jax.experimental import pallas as pl
from jax.experimental.pallas import tpu as pltpu
from jax.experimental.pallas import tpu_sc as plsc

N = 10000
K = 32
D = 128
LANES = 16
NUM_WORKERS = 32
C = 8
CHUNKS = N // C
CW = C * K
MAX_CNT = CHUNKS // NUM_WORKERS + 1
STAGE = MAX_CNT * CW
PAD = 39 * 31 * CW + 2 * CW + STAGE
G = D // LANES


def kernel(x, neighbors, weights):
    nb = neighbors.astype(jnp.int32).reshape(N * K)
    w = weights.reshape(N * K)
    pad = (39 * 31 + 2) * CW + STAGE - N * K
    nb = jnp.pad(nb, (0, pad))
    w = jnp.pad(w, (0, pad))

    mesh = plsc.VectorSubcoreMesh(core_axis_name="core", subcore_axis_name="sub")
    cp = pltpu.CompilerParams()
    if "needs_layout_passes" in pltpu.CompilerParams.__dataclass_fields__:
        cp = dataclasses.replace(cp, needs_layout_passes=False)

    @functools.partial(
        pl.kernel,
        out_type=jax.ShapeDtypeStruct((N, D), jnp.float32),
        mesh=mesh,
        compiler_params=cp,
        scratch_types=[
            pltpu.VMEM((STAGE,), jnp.int32),
            pltpu.VMEM((STAGE,), jnp.float32),
            pltpu.VMEM((2, 128, D), jnp.float32),
            pltpu.VMEM((2, C, D), jnp.float32),
            pltpu.SemaphoreType.DMA,
            pltpu.SemaphoreType.DMA,
            pltpu.SemaphoreType.DMA,
        ],
    )
    def sc_kernel(x_hbm, nb_hbm, w_hbm, out_hbm, idx_v, w_v, rows_v, out_v,
                  gsem, osem0, osem1):
        wid = lax.axis_index("core") * 16 + lax.axis_index("sub")
        s = wid * (CHUNKS // NUM_WORKERS) + jnp.minimum(wid, CHUNKS % NUM_WORKERS)
        cnt = CHUNKS // NUM_WORKERS + (wid < CHUNKS % NUM_WORKERS)

        pltpu.sync_copy(nb_hbm.at[pl.ds(s * CW, STAGE)], idx_v)
        pltpu.sync_copy(w_hbm.at[pl.ds(s * CW, STAGE)], w_v)

        @pl.loop(0, MAX_CNT, step=2)
        def _chunk2(j):
            for b in range(2):
                jj = j + b
                osem = osem0 if b == 0 else osem1

                @pl.when(jj < cnt)
                def _():
                    lo = jj * CW
                    cp0 = pltpu.async_copy(
                        x_hbm.at[idx_v.at[pl.ds(lo, 128)]], rows_v.at[0], gsem)
                    cp1 = pltpu.async_copy(
                        x_hbm.at[idx_v.at[pl.ds(lo + 128, 128)]], rows_v.at[1],
                        gsem)
                    cp0.wait()
                    cp1.wait()

                    @pl.when(jj >= 2)
                    def _():
                        pltpu.make_async_copy(
                            out_v.at[b], out_hbm.at[pl.ds(0, C)], osem).wait()

                    @pl.loop(0, C)
                    def _node(i):
                        half = i // (C // 2)
                        r0 = (i % (C // 2)) * K
                        wv0 = w_v[pl.ds(lo + i * K, LANES)]
                        wv1 = w_v[pl.ds(lo + i * K + LANES, LANES)]
                        total = jnp.sum(wv0 + wv1)
                        tot_v = jnp.broadcast_to(total, (LANES,))
                        zero_v = tot_v == 0.0
                        inv_v = jnp.where(zero_v, jnp.float32(1.0 / K), 1.0 / tot_v)
                        w0n = jnp.where(zero_v, inv_v, wv0 * inv_v)
                        w1n = jnp.where(zero_v, inv_v, wv1 * inv_v)
                        acc = [jnp.zeros((LANES,), jnp.float32) for _ in range(G)]
                        for k in range(K):
                            wk = w0n[k] if k < LANES else w1n[k - LANES]
                            for g in range(G):
                                acc[g] = acc[g] + wk * rows_v[half, r0 + k,
                                                             pl.ds(g * LANES, LANES)]
                        for g in range(G):
                            out_v[b, i, pl.ds(g * LANES, LANES)] = acc[g]

                    pltpu.async_copy(
                        out_v.at[b], out_hbm.at[pl.ds((s + jj) * C, C)], osem)

        for b in range(2):
            osem = osem0 if b == 0 else osem1
            pltpu.make_async_copy(
                out_v.at[b], out_hbm.at[pl.ds(0, C)], osem).wait()

    return sc_kernel(x, nb, w)


# speedup vs baseline: 6.7765x; 1.4547x over previous
import dataclasses
import functools

import jax
import jax.numpy as jnp
from jax import lax
from jax.experimental import pallas as pl
from jax.experimental.pallas import tpu as pltpu
from jax.experimental.pallas import tpu_sc as plsc

N = 10000
K = 32
D = 128
LANES = 16
NUM_WORKERS = 32
C = 8
CHUNKS = N // C
CW = C * K
MAX_CNT = CHUNKS // NUM_WORKERS + 1
STAGE = MAX_CNT * CW
G = D // LANES


def kernel(x, neighbors, weights):
    nb = neighbors.astype(jnp.int32).reshape(N * K)
    w = weights.reshape(N * K)
    pad = (39 * 31 + 2) * CW + STAGE - N * K
    nb = jnp.pad(nb, (0, pad))
    w = jnp.pad(w, (0, pad))

    mesh = plsc.VectorSubcoreMesh(core_axis_name="core", subcore_axis_name="sub")
    cp = pltpu.CompilerParams()
    if "needs_layout_passes" in pltpu.CompilerParams.__dataclass_fields__:
        cp = dataclasses.replace(cp, needs_layout_passes=False)

    @functools.partial(
        pl.kernel,
        out_type=jax.ShapeDtypeStruct((N, D), jnp.float32),
        mesh=mesh,
        compiler_params=cp,
        scratch_types=[
            pltpu.VMEM((STAGE,), jnp.int32),
            pltpu.VMEM((STAGE,), jnp.float32),
            pltpu.VMEM((2, 2, 128, D), jnp.float32),
            pltpu.VMEM((2, C, D), jnp.float32),
            pltpu.SemaphoreType.DMA,
            pltpu.SemaphoreType.DMA,
            pltpu.SemaphoreType.DMA,
            pltpu.SemaphoreType.DMA,
        ],
    )
    def sc_kernel(x_hbm, nb_hbm, w_hbm, out_hbm, idx_v, w_v, rows_v, out_v,
                  gsem0, gsem1, osem0, osem1):
        wid = lax.axis_index("core") * 16 + lax.axis_index("sub")
        s = wid * (CHUNKS // NUM_WORKERS) + jnp.minimum(wid, CHUNKS % NUM_WORKERS)
        cnt = CHUNKS // NUM_WORKERS + (wid < CHUNKS % NUM_WORKERS)

        pltpu.sync_copy(nb_hbm.at[pl.ds(s * CW, STAGE)], idx_v)
        pltpu.sync_copy(w_hbm.at[pl.ds(s * CW, STAGE)], w_v)

        def fire_gathers(jj, buf, gsem):
            lo = jj * CW
            cp0 = pltpu.async_copy(
                x_hbm.at[idx_v.at[pl.ds(lo, 128)]], rows_v.at[buf, 0], gsem)
            cp1 = pltpu.async_copy(
                x_hbm.at[idx_v.at[pl.ds(lo + 128, 128)]], rows_v.at[buf, 1],
                gsem)
            return cp0, cp1

        def compute_chunk(jj, buf, osem):
            lo = jj * CW

            @pl.when(jj >= 2)
            def _():
                pltpu.make_async_copy(
                    out_v.at[buf], out_hbm.at[pl.ds(0, C)], osem).wait()

            @pl.loop(0, C)
            def _node(i):
                half = i // (C // 2)
                r0 = (i % (C // 2)) * K
                wv0 = w_v[pl.ds(lo + i * K, LANES)]
                wv1 = w_v[pl.ds(lo + i * K + LANES, LANES)]
                total = jnp.sum(wv0 + wv1)
                tot_v = jnp.broadcast_to(total, (LANES,))
                zero_v = tot_v == 0.0
                inv_v = jnp.where(zero_v, jnp.float32(1.0 / K), 1.0 / tot_v)
                w0n = jnp.where(zero_v, inv_v, wv0 * inv_v)
                w1n = jnp.where(zero_v, inv_v, wv1 * inv_v)
                acc = [jnp.zeros((LANES,), jnp.float32) for _ in range(G)]
                for k in range(K):
                    wk = w0n[k] if k < LANES else w1n[k - LANES]
                    for g in range(G):
                        acc[g] = acc[g] + wk * rows_v[buf, half, r0 + k,
                                                      pl.ds(g * LANES, LANES)]
                for g in range(G):
                    out_v[buf, i, pl.ds(g * LANES, LANES)] = acc[g]

            pltpu.async_copy(
                out_v.at[buf], out_hbm.at[pl.ds((s + jj) * C, C)], osem)

        @pl.loop(0, MAX_CNT, step=2)
        def _chunk2(j):
            for b in range(2):
                jj = j + b
                osem = osem0 if b == 0 else osem1
                posem = osem1 if b == 0 else osem0
                gsem = gsem0 if b == 0 else gsem1

                @pl.when(jj < cnt)
                def _():
                    cp0, cp1 = fire_gathers(jj, b, gsem)

                    @pl.when(jj >= 1)
                    def _():
                        compute_chunk(jj - 1, 1 - b, posem)

                    cp0.wait()
                    cp1.wait()

        for b in range(2):
            osem = osem0 if b == 0 else osem1

            @pl.when((cnt - 1) % 2 == b)
            def _():
                compute_chunk(cnt - 1, b, osem)

        for b in range(2):
            osem = osem0 if b == 0 else osem1
            pltpu.make_async_copy(
                out_v.at[b], out_hbm.at[pl.ds(0, C)], osem).wait()

    return sc_kernel(x, nb, w)


# speedup vs baseline: 6.8418x; 1.0096x over previous
import dataclasses
import functools

import jax
import jax.numpy as jnp
from jax import lax
from jax.experimental import pallas as pl
from jax.experimental.pallas import tpu as pltpu
from jax.experimental.pallas import tpu_sc as plsc

N = 10000
K = 32
D = 128
LANES = 16
NUM_WORKERS = 32
C = 8
CHUNKS = N // C
CW = C * K
MAX_CNT = CHUNKS // NUM_WORKERS + 1
STAGE = MAX_CNT * CW
G = D // LANES


def kernel(x, neighbors, weights):
    nb = neighbors.astype(jnp.int32).reshape(N * K)
    w = weights.reshape(N * K)
    pad = (39 * 31 + 2) * CW + STAGE - N * K
    nb = jnp.pad(nb, (0, pad))
    w = jnp.pad(w, (0, pad))

    mesh = plsc.VectorSubcoreMesh(core_axis_name="core", subcore_axis_name="sub")
    cp = pltpu.CompilerParams()
    if "needs_layout_passes" in pltpu.CompilerParams.__dataclass_fields__:
        cp = dataclasses.replace(cp, needs_layout_passes=False)

    @functools.partial(
        pl.kernel,
        out_type=jax.ShapeDtypeStruct((N * D,), jnp.float32),
        mesh=mesh,
        compiler_params=cp,
        scratch_types=[
            pltpu.VMEM((STAGE,), jnp.int32),
            pltpu.VMEM((STAGE,), jnp.float32),
            pltpu.VMEM((2, 2, 128, D), jnp.float32),
            pltpu.VMEM((2 * C * D,), jnp.float32),
            pltpu.SemaphoreType.DMA,
            pltpu.SemaphoreType.DMA,
            pltpu.SemaphoreType.DMA,
            pltpu.SemaphoreType.DMA,
        ],
    )
    def sc_kernel(x_hbm, nb_hbm, w_hbm, out_hbm, idx_v, w_v, rows_v, out_v,
                  gsem0, gsem1, osem0, osem1):
        wid = lax.axis_index("core") * 16 + lax.axis_index("sub")
        s = wid * (CHUNKS // NUM_WORKERS) + jnp.minimum(wid, CHUNKS % NUM_WORKERS)
        cnt = CHUNKS // NUM_WORKERS + (wid < CHUNKS % NUM_WORKERS)

        pltpu.sync_copy(nb_hbm.at[pl.ds(s * CW, STAGE)], idx_v)
        pltpu.sync_copy(w_hbm.at[pl.ds(s * CW, STAGE)], w_v)

        def fire_gathers(jj, buf, gsem):
            lo = jj * CW
            cp0 = pltpu.async_copy(
                x_hbm.at[idx_v.at[pl.ds(lo, 128)]], rows_v.at[buf, 0], gsem)
            cp1 = pltpu.async_copy(
                x_hbm.at[idx_v.at[pl.ds(lo + 128, 128)]], rows_v.at[buf, 1],
                gsem)
            return cp0, cp1

        def compute_chunk(jj, buf, osem):
            lo = jj * CW

            @pl.when(jj >= 2)
            def _():
                pltpu.make_async_copy(
                    out_v.at[pl.ds(buf * C * D, C * D)],
                    out_hbm.at[pl.ds(0, C * D)], osem).wait()

            @pl.loop(0, C)
            def _node(i):
                half = i // (C // 2)
                r0 = (i % (C // 2)) * K
                wv0 = w_v[pl.ds(lo + i * K, LANES)]
                wv1 = w_v[pl.ds(lo + i * K + LANES, LANES)]
                total = jnp.sum(wv0 + wv1)
                tot_v = jnp.broadcast_to(total, (LANES,))
                zero_v = tot_v == 0.0
                inv_v = jnp.where(zero_v, jnp.float32(1.0 / K), 1.0 / tot_v)
                w0n = jnp.where(zero_v, inv_v, wv0 * inv_v)
                w1n = jnp.where(zero_v, inv_v, wv1 * inv_v)
                acc = [jnp.zeros((LANES,), jnp.float32) for _ in range(G)]
                for k in range(K):
                    wk = w0n[k] if k < LANES else w1n[k - LANES]
                    for g in range(G):
                        acc[g] = acc[g] + wk * rows_v[buf, half, r0 + k,
                                                      pl.ds(g * LANES, LANES)]
                base = (buf * C + i) * D
                for g in range(G):
                    out_v[pl.ds(base + g * LANES, LANES)] = acc[g]

            pltpu.async_copy(
                out_v.at[pl.ds(buf * C * D, C * D)],
                out_hbm.at[pl.ds((s + jj) * C * D, C * D)], osem)

        @pl.loop(0, MAX_CNT, step=2)
        def _chunk2(j):
            for b in range(2):
                jj = j + b
                osem = osem0 if b == 0 else osem1
                posem = osem1 if b == 0 else osem0
                gsem = gsem0 if b == 0 else gsem1

                @pl.when(jj < cnt)
                def _():
                    cp0, cp1 = fire_gathers(jj, b, gsem)

                    @pl.when(jj >= 1)
                    def _():
                        compute_chunk(jj - 1, 1 - b, posem)

                    cp0.wait()
                    cp1.wait()

        for b in range(2):
            osem = osem0 if b == 0 else osem1

            @pl.when((cnt - 1) % 2 == b)
            def _():
                compute_chunk(cnt - 1, b, osem)

        for b in range(2):
            osem = osem0 if b == 0 else osem1
            pltpu.make_async_copy(
                out_v.at[pl.ds(b * C * D, C * D)],
                out_hbm.at[pl.ds(0, C * D)], osem).wait()

    return sc_kernel(x, nb, w).reshape(N, D)


# speedup vs baseline: 6.9842x; 1.0208x over previous
import dataclasses
import functools

import jax
import jax.numpy as jnp
from jax import lax
from jax.experimental import pallas as pl
from jax.experimental.pallas import tpu as pltpu
from jax.experimental.pallas import tpu_sc as plsc

N = 10000
K = 32
D = 128
LANES = 16
NUM_WORKERS = 32
C = 8
CHUNKS = N // C
CW = C * K
MAX_CNT = CHUNKS // NUM_WORKERS + 1
STAGE = MAX_CNT * CW
G = D // LANES


def kernel(x, neighbors, weights):
    nb = neighbors.astype(jnp.int32).reshape(N * K)
    w = weights.reshape(N * K)

    mesh = plsc.VectorSubcoreMesh(core_axis_name="core", subcore_axis_name="sub")
    cp = pltpu.CompilerParams()
    if "needs_layout_passes" in pltpu.CompilerParams.__dataclass_fields__:
        cp = dataclasses.replace(cp, needs_layout_passes=False)

    @functools.partial(
        pl.kernel,
        out_type=jax.ShapeDtypeStruct((N * D,), jnp.float32),
        mesh=mesh,
        compiler_params=cp,
        scratch_types=[
            pltpu.VMEM((STAGE,), jnp.int32),
            pltpu.VMEM((STAGE,), jnp.float32),
            pltpu.VMEM((2, 2, 128, D), jnp.float32),
            pltpu.VMEM((2 * C * D,), jnp.float32),
            pltpu.SemaphoreType.DMA,
            pltpu.SemaphoreType.DMA,
            pltpu.SemaphoreType.DMA,
            pltpu.SemaphoreType.DMA,
        ],
    )
    def sc_kernel(x_hbm, nb_hbm, w_hbm, out_hbm, idx_v, w_v, rows_v, out_v,
                  gsem0, gsem1, osem0, osem1):
        wid = lax.axis_index("core") * 16 + lax.axis_index("sub")
        s = wid * (CHUNKS // NUM_WORKERS) + jnp.minimum(wid, CHUNKS % NUM_WORKERS)
        cnt = CHUNKS // NUM_WORKERS + (wid < CHUNKS % NUM_WORKERS)

        off = jnp.minimum(s * CW, N * K - STAGE)
        delta = s * CW - off
        pltpu.sync_copy(nb_hbm.at[pl.ds(off, STAGE)], idx_v)
        pltpu.sync_copy(w_hbm.at[pl.ds(off, STAGE)], w_v)

        def fire_gathers(jj, buf, gsem):
            lo = delta + jj * CW
            cp0 = pltpu.async_copy(
                x_hbm.at[idx_v.at[pl.ds(lo, 128)]], rows_v.at[buf, 0], gsem)
            cp1 = pltpu.async_copy(
                x_hbm.at[idx_v.at[pl.ds(lo + 128, 128)]], rows_v.at[buf, 1],
                gsem)
            return cp0, cp1

        def compute_chunk(jj, buf, osem):
            lo = delta + jj * CW

            @pl.when(jj >= 2)
            def _():
                pltpu.make_async_copy(
                    out_v.at[pl.ds(buf * C * D, C * D)],
                    out_hbm.at[pl.ds(0, C * D)], osem).wait()

            @pl.loop(0, C)
            def _node(i):
                half = i // (C // 2)
                r0 = (i % (C // 2)) * K
                wv0 = w_v[pl.ds(lo + i * K, LANES)]
                wv1 = w_v[pl.ds(lo + i * K + LANES, LANES)]
                total = jnp.sum(wv0 + wv1)
                tot_v = jnp.broadcast_to(total, (LANES,))
                zero_v = tot_v == 0.0
                inv_v = jnp.where(zero_v, jnp.float32(1.0 / K), 1.0 / tot_v)
                w0n = jnp.where(zero_v, inv_v, wv0 * inv_v)
                w1n = jnp.where(zero_v, inv_v, wv1 * inv_v)
                acc = [jnp.zeros((LANES,), jnp.float32) for _ in range(G)]
                for k in range(K):
                    wk = w0n[k] if k < LANES else w1n[k - LANES]
                    for g in range(G):
                        acc[g] = acc[g] + wk * rows_v[buf, half, r0 + k,
                                                      pl.ds(g * LANES, LANES)]
                base = (buf * C + i) * D
                for g in range(G):
                    out_v[pl.ds(base + g * LANES, LANES)] = acc[g]

            pltpu.async_copy(
                out_v.at[pl.ds(buf * C * D, C * D)],
                out_hbm.at[pl.ds((s + jj) * C * D, C * D)], osem)

        @pl.loop(0, MAX_CNT, step=2)
        def _chunk2(j):
            for b in range(2):
                jj = j + b
                osem = osem0 if b == 0 else osem1
                posem = osem1 if b == 0 else osem0
                gsem = gsem0 if b == 0 else gsem1

                @pl.when(jj < cnt)
                def _():
                    cp0, cp1 = fire_gathers(jj, b, gsem)

                    @pl.when(jj >= 1)
                    def _():
                        compute_chunk(jj - 1, 1 - b, posem)

                    cp0.wait()
                    cp1.wait()

        for b in range(2):
            osem = osem0 if b == 0 else osem1

            @pl.when((cnt - 1) % 2 == b)
            def _():
                compute_chunk(cnt - 1, b, osem)

        for b in range(2):
            osem = osem0 if b == 0 else osem1
            pltpu.make_async_copy(
                out_v.at[pl.ds(b * C * D, C * D)],
                out_hbm.at[pl.ds(0, C * D)], osem).wait()

    return sc_kernel(x, nb, w).reshape(N, D)


# speedup vs baseline: 7.0284x; 1.0063x over previous
import dataclasses
import functools

import jax
import jax.numpy as jnp
from jax import lax
from jax.experimental import pallas as pl
from jax.experimental.pallas import tpu as pltpu
from jax.experimental.pallas import tpu_sc as plsc

N = 10000
K = 32
D = 128
LANES = 16
NUM_WORKERS = 32
C = 8
CHUNKS = N // C
CW = C * K
MAX_CNT = CHUNKS // NUM_WORKERS + 1
STAGE = MAX_CNT * CW
G = D // LANES


def kernel(x, neighbors, weights):
    nb = neighbors.astype(jnp.int32).reshape(N * K)
    w = weights.reshape(N * K)

    mesh = plsc.VectorSubcoreMesh(core_axis_name="core", subcore_axis_name="sub")
    cp = pltpu.CompilerParams()
    if "needs_layout_passes" in pltpu.CompilerParams.__dataclass_fields__:
        cp = dataclasses.replace(cp, needs_layout_passes=False)

    @functools.partial(
        pl.kernel,
        out_type=jax.ShapeDtypeStruct((N * D,), jnp.float32),
        mesh=mesh,
        compiler_params=cp,
        scratch_types=[
            pltpu.VMEM((STAGE,), jnp.int32),
            pltpu.VMEM((STAGE,), jnp.float32),
            pltpu.VMEM((2, 2, 128, D), jnp.float32),
            pltpu.VMEM((2 * C * D,), jnp.float32),
            pltpu.SemaphoreType.DMA,
            pltpu.SemaphoreType.DMA,
            pltpu.SemaphoreType.DMA,
            pltpu.SemaphoreType.DMA,
        ],
    )
    def sc_kernel(x_hbm, nb_hbm, w_hbm, out_hbm, idx_v, w_v, rows_v, out_v,
                  gsem0, gsem1, osem0, osem1):
        wid = lax.axis_index("sub") * 2 + lax.axis_index("core")
        s = wid * (CHUNKS // NUM_WORKERS) + jnp.minimum(wid, CHUNKS % NUM_WORKERS)
        cnt = CHUNKS // NUM_WORKERS + (wid < CHUNKS % NUM_WORKERS)

        off = jnp.minimum(s * CW, N * K - STAGE)
        delta = s * CW - off
        pltpu.sync_copy(nb_hbm.at[pl.ds(off, STAGE)], idx_v)
        pltpu.sync_copy(w_hbm.at[pl.ds(off, STAGE)], w_v)

        def fire_gathers(jj, buf, gsem):
            lo = delta + jj * CW
            cp0 = pltpu.async_copy(
                x_hbm.at[idx_v.at[pl.ds(lo, 128)]], rows_v.at[buf, 0], gsem)
            cp1 = pltpu.async_copy(
                x_hbm.at[idx_v.at[pl.ds(lo + 128, 128)]], rows_v.at[buf, 1],
                gsem)
            return cp0, cp1

        def compute_chunk(jj, buf, osem):
            lo = delta + jj * CW

            @pl.when(jj >= 2)
            def _():
                pltpu.make_async_copy(
                    out_v.at[pl.ds(buf * C * D, C * D)],
                    out_hbm.at[pl.ds(0, C * D)], osem).wait()

            @pl.loop(0, C)
            def _node(i):
                half = i // (C // 2)
                r0 = (i % (C // 2)) * K
                wv0 = w_v[pl.ds(lo + i * K, LANES)]
                wv1 = w_v[pl.ds(lo + i * K + LANES, LANES)]
                total = jnp.sum(wv0 + wv1)
                tot_v = jnp.broadcast_to(total, (LANES,))
                zero_v = tot_v == 0.0
                inv_v = jnp.where(zero_v, jnp.float32(1.0 / K), 1.0 / tot_v)
                w0n = jnp.where(zero_v, inv_v, wv0 * inv_v)
                w1n = jnp.where(zero_v, inv_v, wv1 * inv_v)
                acc = [jnp.zeros((LANES,), jnp.float32) for _ in range(G)]
                for k in range(K):
                    wk = w0n[k] if k < LANES else w1n[k - LANES]
                    for g in range(G):
                        acc[g] = acc[g] + wk * rows_v[buf, half, r0 + k,
                                                      pl.ds(g * LANES, LANES)]
                base = (buf * C + i) * D
                for g in range(G):
                    out_v[pl.ds(base + g * LANES, LANES)] = acc[g]

            pltpu.async_copy(
                out_v.at[pl.ds(buf * C * D, C * D)],
                out_hbm.at[pl.ds((s + jj) * C * D, C * D)], osem)

        @pl.loop(0, MAX_CNT, step=2)
        def _chunk2(j):
            for b in range(2):
                jj = j + b
                osem = osem0 if b == 0 else osem1
                posem = osem1 if b == 0 else osem0
                gsem = gsem0 if b == 0 else gsem1

                @pl.when(jj < cnt)
                def _():
                    cp0, cp1 = fire_gathers(jj, b, gsem)

                    @pl.when(jj >= 1)
                    def _():
                        compute_chunk(jj - 1, 1 - b, posem)

                    cp0.wait()
                    cp1.wait()

        for b in range(2):
            osem = osem0 if b == 0 else osem1

            @pl.when((cnt - 1) % 2 == b)
            def _():
                compute_chunk(cnt - 1, b, osem)

        for b in range(2):
            osem = osem0 if b == 0 else osem1
            pltpu.make_async_copy(
                out_v.at[pl.ds(b * C * D, C * D)],
                out_hbm.at[pl.ds(0, C * D)], osem).wait()

    return sc_kernel(x, nb, w).reshape(N, D)
